# baseline (speedup 1.0000x reference)
import functools

import jax
import jax.numpy as jnp
from jax import lax
from jax.experimental import pallas as pl
from jax.experimental.pallas import tpu as pltpu
from jax.experimental.pallas import tpu_sc as plsc

VOCAB = 100
EMB = 8
B = 16384
L = 200
TOK = B * L

NC, NS = 2, 16
NW = NC * NS
PER_W = TOK // NW
GRP = 16
BLK = 1024
NBLK = PER_W // BLK
GPB = BLK // GRP


def _fuse_table_body(emb_ref, w_ref, b_ref, out_ref):
    out_ref[...] = lax.dot_general(
        emb_ref[...], w_ref[...],
        dimension_numbers=(((1,), (1,)), ((), ())),
        preferred_element_type=jnp.float32) + b_ref[...]


def _fused_table(emb_table, W, b):
    return pl.pallas_call(
        _fuse_table_body,
        out_shape=jax.ShapeDtypeStruct((VOCAB, EMB), jnp.float32),
    )(emb_table, W, b.reshape(1, EMB))


_sc_mesh = plsc.VectorSubcoreMesh(core_axis_name="c", subcore_axis_name="s")


@functools.partial(
    pl.kernel,
    out_type=jax.ShapeDtypeStruct((TOK * EMB,), jnp.float32),
    mesh=_sc_mesh,
    scratch_types=[
        pltpu.VMEM((VOCAB * EMB,), jnp.float32),
        pltpu.VMEM((PER_W,), jnp.int32),
        pltpu.VMEM((BLK * EMB,), jnp.float32),
        pltpu.VMEM((BLK * EMB,), jnp.float32),
        pltpu.SemaphoreType.DMA,
        pltpu.SemaphoreType.DMA,
    ],
    compiler_params=pltpu.CompilerParams(
        use_tc_tiling_on_sc=False, needs_layout_passes=False),
)
def _gather_kernel(table_hbm, ids_hbm, out_hbm, table_v, ids_v,
                   stg_a, stg_b, sem_a, sem_b):
    wid = lax.axis_index("s") * NC + lax.axis_index("c")
    tok0 = wid * PER_W

    pltpu.sync_copy(table_hbm, table_v)
    pltpu.sync_copy(ids_hbm.at[pl.ds(tok0, PER_W)], ids_v)

    iota = lax.iota(jnp.int32, GRP)
    cols = [iota * EMB + j for j in range(EMB)]
    bufs = ((stg_a, sem_a), (stg_b, sem_b))

    def fill(blk, stg_v):
        def group(g, _):
            ids = ids_v[pl.ds((blk * GPB + g) * GRP, GRP)]
            flat = ids * EMB
            ob = stg_v.at[pl.ds(g * (GRP * EMB), GRP * EMB)]
            for j in range(EMB):
                col = plsc.load_gather(table_v, [flat + j])
                plsc.store_scatter(ob, [cols[j]], col)
            return ()

        lax.fori_loop(0, GPB, group, (), unroll=8)

    def out_window(blk):
        return out_hbm.at[pl.ds((tok0 + blk * BLK) * EMB, BLK * EMB)]

    for p, (stg_v, sem) in enumerate(bufs):
        fill(p, stg_v)
        pltpu.async_copy(stg_v, out_window(p), sem)

    def pair(i, _):
        for p, (stg_v, sem) in enumerate(bufs):
            blk = 2 * i + p
            pltpu.make_async_copy(stg_v, out_window(blk), sem).wait()
            fill(blk, stg_v)
            pltpu.async_copy(stg_v, out_window(blk), sem)
        return ()

    lax.fori_loop(1, NBLK // 2, pair, ())

    for p, (stg_v, sem) in enumerate(bufs):
        pltpu.make_async_copy(stg_v, out_window(NBLK - 2 + p), sem).wait()


def kernel(input_ids, emb_table, W, b):
    table = _fused_table(emb_table, W, b).reshape(VOCAB * EMB)
    ids = input_ids.reshape(TOK).astype(jnp.int32)
    flat = _gather_kernel(table, ids)
    one = (b[0] - b[0]) + jnp.float32(1.0)
    return flat.reshape(B, L, EMB) * one


# speedup vs baseline: 1.4487x; 1.4487x over previous
import functools

import jax
import jax.numpy as jnp
from jax import lax
from jax.experimental import pallas as pl
from jax.experimental.pallas import tpu as pltpu
from jax.experimental.pallas import tpu_sc as plsc

VOCAB = 100
EMB = 8
B = 16384
L = 200
TOK = B * L

NC, NS = 2, 16
NW = NC * NS
ROWS_W = B // NW
NB = 2
BLK = NB * L
NBLK = ROWS_W // NB
NPAIR = NBLK // 2
GRP = 16
GPB = BLK // GRP


def _fuse_table_body(emb_ref, w_ref, b_ref, out_ref):
    out_ref[...] = lax.dot_general(
        emb_ref[...], w_ref[...],
        dimension_numbers=(((1,), (1,)), ((), ())),
        preferred_element_type=jnp.float32) + b_ref[...]


def _fused_table(emb_table, W, b):
    return pl.pallas_call(
        _fuse_table_body,
        out_shape=jax.ShapeDtypeStruct((VOCAB, EMB), jnp.float32),
    )(emb_table, W, b.reshape(1, EMB))


_sc_mesh = plsc.VectorSubcoreMesh(core_axis_name="c", subcore_axis_name="s")


@functools.partial(
    pl.kernel,
    out_type=jax.ShapeDtypeStruct((B, L, EMB), jnp.float32),
    mesh=_sc_mesh,
    scratch_types=[
        pltpu.VMEM((VOCAB * EMB,), jnp.float32),
        pltpu.VMEM((BLK,), jnp.int32),
        pltpu.VMEM((BLK,), jnp.int32),
        pltpu.VMEM((NB, L, EMB), jnp.float32),
        pltpu.VMEM((NB, L, EMB), jnp.float32),
        pltpu.SemaphoreType.DMA,
        pltpu.SemaphoreType.DMA,
        pltpu.SemaphoreType.DMA,
        pltpu.SemaphoreType.DMA,
    ],
    compiler_params=pltpu.CompilerParams(needs_layout_passes=False),
)
def _gather_kernel(table_hbm, ids_hbm, out_hbm, table_v,
                   ids_a, ids_b, stg_a, stg_b, sem_a, sem_b, isem_a, isem_b):
    wid = lax.axis_index("s") * NC + lax.axis_index("c")
    row0 = wid * ROWS_W

    pltpu.sync_copy(table_hbm, table_v)

    iota = lax.iota(jnp.int32, GRP)
    cols = [jnp.full((GRP,), j, jnp.int32) for j in range(EMB)]
    bufs = ((ids_a, stg_a, sem_a, isem_a), (ids_b, stg_b, sem_b, isem_b))

    def ids_window(blk):
        return ids_hbm.at[pl.ds((row0 + blk * NB) * L, BLK)]

    def out_window(blk):
        return out_hbm.at[pl.ds(row0 + blk * NB, NB), :, :]

    def fill(ids_v, stg_v):
        def group(g, _):
            ids = ids_v[pl.ds(g * GRP, GRP)]
            flat = ids * EMB
            t = iota + g * GRP
            bl = t // L
            ll = t - bl * L
            for j in range(EMB):
                col = plsc.load_gather(table_v, [flat + j])
                plsc.store_scatter(stg_v, [bl, ll, cols[j]], col)
            return ()

        lax.fori_loop(0, GPB, group, (), unroll=5)

    for p, (ids_v, _, _, isem) in enumerate(bufs):
        pltpu.async_copy(ids_window(p), ids_v, isem)
    for p, (ids_v, stg_v, sem, isem) in enumerate(bufs):
        pltpu.make_async_copy(ids_window(p), ids_v, isem).wait()
        fill(ids_v, stg_v)
        pltpu.async_copy(stg_v, out_window(p), sem)
        pltpu.async_copy(ids_window(p + 2), ids_v, isem)

    def pair(i, _):
        for p, (ids_v, stg_v, sem, isem) in enumerate(bufs):
            blk = 2 * i + p
            pltpu.make_async_copy(ids_window(blk), ids_v, isem).wait()
            pltpu.make_async_copy(stg_v, out_window(blk), sem).wait()
            fill(ids_v, stg_v)
            pltpu.async_copy(stg_v, out_window(blk), sem)
            nxt = jnp.minimum(blk + 2, NBLK - 2 + p)
            pltpu.async_copy(ids_window(nxt), ids_v, isem)
        return ()

    lax.fori_loop(1, NPAIR, pair, ())

    for p, (ids_v, stg_v, sem, isem) in enumerate(bufs):
        pltpu.make_async_copy(stg_v, out_window(NBLK - 2 + p), sem).wait()
        pltpu.make_async_copy(ids_window(p), ids_v, isem).wait()


def kernel(input_ids, emb_table, W, b):
    table = _fused_table(emb_table, W, b).reshape(VOCAB * EMB)
    ids = input_ids.reshape(TOK).astype(jnp.int32)
    return _gather_kernel(table, ids)
